# baseline (speedup 1.0000x reference)
import functools

import jax
import jax.numpy as jnp
from jax import lax
from jax.experimental import pallas as pl
from jax.experimental.pallas import tpu as pltpu
from jax.experimental.pallas import tpu_sc as plsc

B = 32
D = 512
V = 50257

_ROWS_PER_WORKER = 8
_ACTIVE_WORKERS = B // _ROWS_PER_WORKER

_sc_mesh = plsc.VectorSubcoreMesh(core_axis_name="c", subcore_axis_name="s")


@functools.partial(
    pl.kernel,
    out_type=jax.ShapeDtypeStruct((B, D), jnp.float32),
    mesh=_sc_mesh,
    scratch_types=[
        pltpu.VMEM((_ROWS_PER_WORKER,), jnp.int32),
        pltpu.VMEM((_ROWS_PER_WORKER, D), jnp.float32),
        pltpu.SemaphoreType.DMA,
    ],
)
def _sc_gather(emb_hbm, ids_hbm, out_hbm, idx_v, rows_v, sem):
    info = plsc.get_sparse_core_info()
    nc = info.num_cores
    wid = lax.axis_index("s") * nc + lax.axis_index("c")

    @pl.when(wid < _ACTIVE_WORKERS)
    def _():
        base = wid * _ROWS_PER_WORKER
        pltpu.sync_copy(ids_hbm.at[pl.ds(base, _ROWS_PER_WORKER)], idx_v)
        pltpu.async_copy(emb_hbm.at[idx_v], rows_v, sem).wait()
        pltpu.sync_copy(rows_v, out_hbm.at[pl.ds(base, _ROWS_PER_WORKER)])


_VB = 2048
_NV = (V + _VB - 1) // _VB
_TAIL = V - (_NV - 1) * _VB
_NBUF = 4


def _w_full(w_hbm, bufs, sems, slot, bv):
    return pltpu.make_async_copy(
        w_hbm.at[:, pl.ds(bv * _VB, _VB)], bufs.at[slot], sems.at[slot]
    )


def _mm_body(x_ref, b_ref, wt_ref, w_hbm, o_ref, bufs, sems):
    v = pl.program_id(0)

    @pl.when(v == 0)
    def _():
        for i in range(_NBUF):
            _w_full(w_hbm, bufs, sems, i, i).start()

    @pl.when(v < _NV - 1)
    def _():
        slot = lax.rem(v, _NBUF)
        for s in range(_NBUF):

            @pl.when(slot == s)
            def _(s=s):
                _w_full(w_hbm, bufs, sems, s, v).wait()
                o_ref[...] = (
                    jnp.dot(
                        x_ref[...], bufs[s], preferred_element_type=jnp.float32
                    )
                    + b_ref[...]
                )
                nxt = v + _NBUF

                @pl.when(nxt < _NV - 1)
                def _():
                    _w_full(w_hbm, bufs, sems, s, nxt).start()

    @pl.when(v == _NV - 1)
    def _():
        o_ref[...] = (
            jnp.dot(x_ref[...], wt_ref[...], preferred_element_type=jnp.float32)
            + b_ref[...]
        )


def _tc_project(x, W, b2d):
    return pl.pallas_call(
        _mm_body,
        grid=(_NV,),
        in_specs=[
            pl.BlockSpec((B, D), lambda v: (0, 0)),
            pl.BlockSpec((1, _VB), lambda v: (0, v)),
            pl.BlockSpec((D, _VB), lambda v: (0, _NV - 1)),
            pl.BlockSpec(memory_space=pltpu.MemorySpace.HBM),
        ],
        out_specs=pl.BlockSpec((B, _VB), lambda v: (0, v)),
        out_shape=jax.ShapeDtypeStruct((B, V), jnp.float32),
        scratch_shapes=[
            pltpu.VMEM((_NBUF, D, _VB), jnp.float32),
            pltpu.SemaphoreType.DMA((_NBUF,)),
        ],
        compiler_params=pltpu.CompilerParams(
            dimension_semantics=("arbitrary",),
        ),
    )(x, b2d, W, W)


def kernel(input_ids, emb, W, b):
    ids = input_ids.reshape(B).astype(jnp.int32)
    x = _sc_gather(emb, ids)
    logits = _tc_project(x, W, b.reshape(1, V))
    return logits.reshape(B, 1, V)
